# Initial kernel scaffold; baseline (speedup 1.0000x reference)
import jax
import jax.numpy as jnp
from jax.experimental import pallas as pl


def kernel(x, batch, original_coords, step_count, num_layer, W_pre1, b_pre1, W_pre2, b_pre2, g1, be1, W_s, W_h, b_h, W_lin, b_lin, W_post1, b_post1, W_post2, b_post2, g2, be2):
    raise NotImplementedError("write your pallas kernel here")


# speedup vs baseline: 5.5044x; 5.5044x over previous
import functools

import jax
import jax.numpy as jnp
import numpy as np
from jax import lax
from jax.experimental import pallas as pl
from jax.experimental.pallas import tpu as pltpu
from jax.experimental.pallas import tpu_sc as plsc

N = 10000
NP = 10240
D = 32
K = 40
BIG = np.float32(3.0e38)

_IP = False


def _elu(v):
    return jnp.where(v > 0, v, jnp.exp(jnp.minimum(v, 0.0)) - 1.0)


def _pre_body(x_ref, w1_ref, b1_ref, w2_ref, b2_ref, h2_ref, sums_ref):
    i = pl.program_id(0)
    h = jnp.dot(x_ref[...], w1_ref[...], preferred_element_type=jnp.float32)
    h = _elu(h + b1_ref[...])
    h = jnp.dot(h, w2_ref[...], preferred_element_type=jnp.float32)
    h = _elu(h + b2_ref[...])
    h2_ref[...] = h
    br = h.shape[0]
    rows = i * br + lax.broadcasted_iota(jnp.int32, (br, 1), 0)
    hm = jnp.where(rows < N, h, 0.0)

    @pl.when(i == 0)
    def _():
        sums_ref[...] = jnp.zeros_like(sums_ref)

    sums_ref[0:1, :] += jnp.sum(hm, axis=0, keepdims=True)
    sums_ref[1:2, :] += jnp.sum(hm * hm, axis=0, keepdims=True)


def _pre_mlp(xp, W1, b1, W2, b2):
    br = 512
    grid = NP // br
    return pl.pallas_call(
        _pre_body,
        grid=grid,
        in_specs=[
            pl.BlockSpec((br, 256), lambda i: (i, 0)),
            pl.BlockSpec((256, D), lambda i: (0, 0)),
            pl.BlockSpec((1, D), lambda i: (0, 0)),
            pl.BlockSpec((D, D), lambda i: (0, 0)),
            pl.BlockSpec((1, D), lambda i: (0, 0)),
        ],
        out_specs=[
            pl.BlockSpec((br, D), lambda i: (i, 0)),
            pl.BlockSpec((2, D), lambda i: (0, 0)),
        ],
        out_shape=[
            jax.ShapeDtypeStruct((NP, D), jnp.float32),
            jax.ShapeDtypeStruct((2, D), jnp.float32),
        ],
        interpret=_IP,
    )(xp, W1, b1, W2, b2)


def _proj_body(h2_ref, a_ref, c_ref, wh_ref, bh_ref, ws_ref,
               bn_ref, hl_ref, srows_ref):
    bn = h2_ref[...] * a_ref[...] + c_ref[...]
    bn_ref[...] = bn
    hl_ref[...] = jnp.dot(bn, wh_ref[...],
                          preferred_element_type=jnp.float32) + bh_ref[...]
    s8 = jnp.dot(bn, ws_ref[...], preferred_element_type=jnp.float32)
    ss = jnp.sum(s8 * s8, axis=1, keepdims=True)
    br = bn.shape[0]
    z1 = jnp.zeros((br, 1), jnp.float32)
    srows_ref[...] = jnp.concatenate(
        [s8[:, 0:3], z1, ss, jnp.zeros((br, 3), jnp.float32)], axis=1)


def _projections(h2, a1, c1, Wh, bh, Ws8):
    br = 512
    grid = NP // br
    return pl.pallas_call(
        _proj_body,
        grid=grid,
        in_specs=[
            pl.BlockSpec((br, D), lambda i: (i, 0)),
            pl.BlockSpec((1, D), lambda i: (0, 0)),
            pl.BlockSpec((1, D), lambda i: (0, 0)),
            pl.BlockSpec((D, D), lambda i: (0, 0)),
            pl.BlockSpec((1, D), lambda i: (0, 0)),
            pl.BlockSpec((D, 8), lambda i: (0, 0)),
        ],
        out_specs=[
            pl.BlockSpec((br, D), lambda i: (i, 0)),
            pl.BlockSpec((br, D), lambda i: (i, 0)),
            pl.BlockSpec((br, 8), lambda i: (i, 0)),
        ],
        out_shape=[
            jax.ShapeDtypeStruct((NP, D), jnp.float32),
            jax.ShapeDtypeStruct((NP, D), jnp.float32),
            jax.ShapeDtypeStruct((NP, 8), jnp.float32),
        ],
        interpret=_IP,
    )(h2, a1, c1, Wh, bh, Ws8)


def _knn_body(srows_ref, sT_ref, idx_ref, d2_ref):
    i = pl.program_id(0)
    br = srows_ref.shape[0]
    d2 = (srows_ref[:, 4:5] + sT_ref[4:5, :]
          - 2.0 * jnp.dot(srows_ref[:, 0:4], sT_ref[0:4, :],
                          preferred_element_type=jnp.float32))
    rows = i * br + lax.broadcasted_iota(jnp.int32, (br, NP), 0)
    cols = lax.broadcasted_iota(jnp.int32, (br, NP), 1)
    d2 = jnp.where(cols == rows, BIG, d2)
    d2_ref[...] = d2
    for t in range(K):
        d2 = d2_ref[...]
        m = jnp.min(d2, axis=1, keepdims=True)
        j = jnp.min(jnp.where(d2 == m, cols, np.int32(2 ** 30)),
                    axis=1, keepdims=True)
        idx_ref[:, t:t + 1] = j
        d2_ref[...] = jnp.where(cols == j, BIG, d2)


def _knn(srows, sT):
    br = 256
    grid = NP // br
    return pl.pallas_call(
        _knn_body,
        grid=grid,
        in_specs=[
            pl.BlockSpec((br, 8), lambda i: (i, 0)),
            pl.BlockSpec((8, NP), lambda i: (0, 0)),
        ],
        out_specs=pl.BlockSpec((br, K), lambda i: (i, 0)),
        out_shape=jax.ShapeDtypeStruct((NP, K), jnp.int32),
        scratch_shapes=[pltpu.VMEM((br, NP), jnp.float32)],
        interpret=_IP,
    )(srows, sT)


_NW = 32
_BPW = NP // _NW
_CH = 32
_NCH = _BPW // _CH


def _sc_agg_body(idx_hbm, stab_hbm, s16_hbm, hl_hbm, out_hbm,
                 xs_v, ys_v, zs_v, ss_v, idx_v, rows_v, s_v, agg_v, sem):
    wid = lax.axis_index("s") * 2 + lax.axis_index("c")
    pltpu.sync_copy(stab_hbm.at[0], xs_v)
    pltpu.sync_copy(stab_hbm.at[1], ys_v)
    pltpu.sync_copy(stab_hbm.at[2], zs_v)
    pltpu.sync_copy(stab_hbm.at[3], ss_v)

    def chunk_body(c, _):
        base = wid * _BPW + c * _CH
        pltpu.sync_copy(idx_hbm.at[pl.ds(base * K, _CH * K)], idx_v)
        pltpu.sync_copy(s16_hbm.at[pl.ds(base, _CH)], s_v)
        descs = [
            pltpu.async_copy(hl_hbm.at[idx_v.at[pl.ds(g * 128, 128)]],
                             rows_v.at[pl.ds(g * 128, 128)], sem)
            for g in range(_CH * K // 128)
        ]
        for dsc in descs:
            dsc.wait()

        def dst_body(di, _):
            svec = s_v[di, 0:16]
            si0, si1, si2, ssi = svec[0], svec[1], svec[2], svec[4]
            pots = []
            for off in (0, 16, 24):
                jg = idx_v[pl.ds(di * K + off, 16)]
                sjx = plsc.load_gather(xs_v, [jg])
                sjy = plsc.load_gather(ys_v, [jg])
                sjz = plsc.load_gather(zs_v, [jg])
                sjs = plsc.load_gather(ss_v, [jg])
                dd = ssi + sjs - 2.0 * (si0 * sjx + si1 * sjy + si2 * sjz)
                pots.append(jnp.exp(-jnp.maximum(dd, 0.0)))
            pa, pb, pc = pots
            z = jnp.zeros((16,), jnp.float32)
            neg = jnp.full((16,), -3.0e38, jnp.float32)
            m0, m1, x0, x1 = z, z, neg, neg
            for k in range(K):
                p = pa[k] if k < 16 else (pb[k - 16] if k < 32 else pc[k - 24])
                r = di * K + k
                f0 = p * rows_v[r, 0:16]
                f1 = p * rows_v[r, 16:32]
                m0 = m0 + f0
                m1 = m1 + f1
                x0 = jnp.maximum(x0, f0)
                x1 = jnp.maximum(x1, f1)
            agg_v[di, 0:16] = m0 / 40.0
            agg_v[di, 16:32] = m1 / 40.0
            agg_v[di, 32:48] = x0
            agg_v[di, 48:64] = x1
            return 0

        lax.fori_loop(0, _CH, dst_body, 0)
        pltpu.sync_copy(agg_v, out_hbm.at[pl.ds(base, _CH)])
        return 0

    lax.fori_loop(0, _NCH, chunk_body, 0)


def _sc_aggregate(idx_flat, stab, s16, hl):
    mesh = plsc.VectorSubcoreMesh(core_axis_name="c", subcore_axis_name="s")
    f = pl.kernel(
        _sc_agg_body,
        out_type=jax.ShapeDtypeStruct((NP, 2 * D), jnp.float32),
        mesh=mesh,
        scratch_types=[
            pltpu.VMEM((NP,), jnp.float32),
            pltpu.VMEM((NP,), jnp.float32),
            pltpu.VMEM((NP,), jnp.float32),
            pltpu.VMEM((NP,), jnp.float32),
            pltpu.VMEM((_CH * K,), jnp.int32),
            pltpu.VMEM((_CH * K, D), jnp.float32),
            pltpu.VMEM((_CH, 16), jnp.float32),
            pltpu.VMEM((_CH, 2 * D), jnp.float32),
            pltpu.SemaphoreType.DMA,
        ],
        compiler_params=pltpu.CompilerParams(use_tc_tiling_on_sc=False,
                                             needs_layout_passes=False),
    )
    return f(idx_flat, stab, s16, hl)


def _post_body(agg_ref, bn_ref, srows_ref, wla_ref, wlh_ref, bl_ref,
               wp1x_ref, wp1s_ref, wp1h_ref, bp1_ref, wp2_ref, bp2_ref,
               z2_ref, sums_ref):
    i = pl.program_id(0)
    xgn = (jnp.dot(agg_ref[...], wla_ref[...],
                   preferred_element_type=jnp.float32)
           + jnp.dot(bn_ref[...], wlh_ref[...],
                     preferred_element_type=jnp.float32)
           + bl_ref[...])
    z = (jnp.dot(xgn, wp1x_ref[...], preferred_element_type=jnp.float32)
         + jnp.dot(srows_ref[...], wp1s_ref[...],
                   preferred_element_type=jnp.float32)
         + jnp.dot(bn_ref[...], wp1h_ref[...],
                   preferred_element_type=jnp.float32)
         + bp1_ref[...])
    z = _elu(z)
    z = jnp.dot(z, wp2_ref[...], preferred_element_type=jnp.float32)
    z = _elu(z + bp2_ref[...])
    z2_ref[...] = z
    br = z.shape[0]
    rows = i * br + lax.broadcasted_iota(jnp.int32, (br, 1), 0)
    zm = jnp.where(rows < N, z, 0.0)

    @pl.when(i == 0)
    def _():
        sums_ref[...] = jnp.zeros_like(sums_ref)

    sums_ref[0:1, :] += jnp.sum(zm, axis=0, keepdims=True)
    sums_ref[1:2, :] += jnp.sum(zm * zm, axis=0, keepdims=True)


def _post_mlp(agg, bn, srows, Wla, Wlh, bl, Wp1x, Wp1s, Wp1h, bp1, Wp2, bp2):
    br = 512
    grid = NP // br
    return pl.pallas_call(
        _post_body,
        grid=grid,
        in_specs=[
            pl.BlockSpec((br, 2 * D), lambda i: (i, 0)),
            pl.BlockSpec((br, D), lambda i: (i, 0)),
            pl.BlockSpec((br, 8), lambda i: (i, 0)),
            pl.BlockSpec((2 * D, D), lambda i: (0, 0)),
            pl.BlockSpec((D, D), lambda i: (0, 0)),
            pl.BlockSpec((1, D), lambda i: (0, 0)),
            pl.BlockSpec((D, D), lambda i: (0, 0)),
            pl.BlockSpec((8, D), lambda i: (0, 0)),
            pl.BlockSpec((D, D), lambda i: (0, 0)),
            pl.BlockSpec((1, D), lambda i: (0, 0)),
            pl.BlockSpec((D, D), lambda i: (0, 0)),
            pl.BlockSpec((1, D), lambda i: (0, 0)),
        ],
        out_specs=[
            pl.BlockSpec((br, D), lambda i: (i, 0)),
            pl.BlockSpec((2, D), lambda i: (0, 0)),
        ],
        out_shape=[
            jax.ShapeDtypeStruct((NP, D), jnp.float32),
            jax.ShapeDtypeStruct((2, D), jnp.float32),
        ],
        interpret=_IP,
    )(agg, bn, srows, Wla, Wlh, bl, Wp1x, Wp1s, Wp1h, bp1, Wp2, bp2)


def _aff_body(z_ref, a_ref, c_ref, o_ref):
    o_ref[...] = z_ref[...] * a_ref[...] + c_ref[...]


def _affine(z2, a2, c2):
    br = 1024
    grid = NP // br
    return pl.pallas_call(
        _aff_body,
        grid=grid,
        in_specs=[
            pl.BlockSpec((br, D), lambda i: (i, 0)),
            pl.BlockSpec((1, D), lambda i: (0, 0)),
            pl.BlockSpec((1, D), lambda i: (0, 0)),
        ],
        out_specs=pl.BlockSpec((br, D), lambda i: (i, 0)),
        out_shape=jax.ShapeDtypeStruct((NP, D), jnp.float32),
        interpret=_IP,
    )(z2, a2, c2)


def kernel(x, batch, original_coords, step_count, num_layer,
           W_pre1, b_pre1, W_pre2, b_pre2, g1, be1,
           W_s, W_h, b_h, W_lin, b_lin,
           W_post1, b_post1, W_post2, b_post2, g2, be2):
    eps = 1e-5
    xp = jnp.pad(x, ((0, NP - N), (0, 0)))
    r1 = lambda v: v.reshape(1, D)

    h2, sums1 = _pre_mlp(xp, W_pre1, r1(b_pre1), W_pre2, r1(b_pre2))
    mu1 = sums1[0] / N
    var1 = sums1[1] / N - mu1 * mu1
    a1 = g1 / jnp.sqrt(var1 + eps)
    c1 = be1 - mu1 * a1

    Ws8 = jnp.pad(W_s, ((0, 0), (0, 5)))
    bn, hl, srows = _projections(h2, a1.reshape(1, D), c1.reshape(1, D),
                                 W_h, r1(b_h), Ws8)

    sT = srows[:, 0:8].T
    colv = jnp.arange(NP) < N
    sT = sT.at[4, :].set(jnp.where(colv, sT[4, :], 3.0e38))

    nn_idx = _knn(srows, sT)

    stab = jnp.stack([srows[:, 0], srows[:, 1], srows[:, 2], srows[:, 4]])
    s16 = jnp.pad(srows, ((0, 0), (0, 8)))
    agg = _sc_aggregate(nn_idx.reshape(-1), stab, s16, hl)

    Wla, Wlh = W_lin[:2 * D], W_lin[2 * D:]
    Wp1x = W_post1[0:D]
    Wp1s = jnp.pad(W_post1[D:D + 3], ((0, 5), (0, 0)))
    Wp1h = W_post1[D + 3:]
    z2, sums2 = _post_mlp(agg, bn, srows, Wla, Wlh, r1(b_lin),
                          Wp1x, Wp1s, Wp1h, r1(b_post1), W_post2,
                          r1(b_post2))
    mu2 = sums2[0] / N
    var2 = sums2[1] / N - mu2 * mu2
    a2 = g2 / jnp.sqrt(var2 + eps)
    c2 = be2 - mu2 * a2

    out = _affine(z2, a2.reshape(1, D), c2.reshape(1, D))
    return out[:N]


# speedup vs baseline: 5.7447x; 1.0437x over previous
import functools

import jax
import jax.numpy as jnp
import numpy as np
from jax import lax
from jax.experimental import pallas as pl
from jax.experimental.pallas import tpu as pltpu
from jax.experimental.pallas import tpu_sc as plsc

N = 10000
NP = 10240
D = 32
K = 40
BIG = np.float32(3.0e38)

_IP = False


def _elu(v):
    return jnp.where(v > 0, v, jnp.exp(jnp.minimum(v, 0.0)) - 1.0)


def _pre_body(x_ref, w1_ref, b1_ref, w2_ref, b2_ref, h2_ref, sums_ref):
    i = pl.program_id(0)
    h = jnp.dot(x_ref[...], w1_ref[...], preferred_element_type=jnp.float32)
    h = _elu(h + b1_ref[...])
    h = jnp.dot(h, w2_ref[...], preferred_element_type=jnp.float32)
    h = _elu(h + b2_ref[...])
    h2_ref[...] = h
    br = h.shape[0]
    rows = i * br + lax.broadcasted_iota(jnp.int32, (br, 1), 0)
    hm = jnp.where(rows < N, h, 0.0)

    @pl.when(i == 0)
    def _():
        sums_ref[...] = jnp.zeros_like(sums_ref)

    sums_ref[0:1, :] += jnp.sum(hm, axis=0, keepdims=True)
    sums_ref[1:2, :] += jnp.sum(hm * hm, axis=0, keepdims=True)


def _pre_mlp(xp, W1, b1, W2, b2):
    br = 512
    grid = NP // br
    return pl.pallas_call(
        _pre_body,
        grid=grid,
        in_specs=[
            pl.BlockSpec((br, 256), lambda i: (i, 0)),
            pl.BlockSpec((256, D), lambda i: (0, 0)),
            pl.BlockSpec((1, D), lambda i: (0, 0)),
            pl.BlockSpec((D, D), lambda i: (0, 0)),
            pl.BlockSpec((1, D), lambda i: (0, 0)),
        ],
        out_specs=[
            pl.BlockSpec((br, D), lambda i: (i, 0)),
            pl.BlockSpec((2, D), lambda i: (0, 0)),
        ],
        out_shape=[
            jax.ShapeDtypeStruct((NP, D), jnp.float32),
            jax.ShapeDtypeStruct((2, D), jnp.float32),
        ],
        interpret=_IP,
    )(xp, W1, b1, W2, b2)


def _proj_body(h2_ref, a_ref, c_ref, wh_ref, bh_ref, ws_ref,
               bn_ref, hl_ref, srows_ref):
    bn = h2_ref[...] * a_ref[...] + c_ref[...]
    bn_ref[...] = bn
    hl_ref[...] = jnp.dot(bn, wh_ref[...],
                          preferred_element_type=jnp.float32) + bh_ref[...]
    s8 = jnp.dot(bn, ws_ref[...], preferred_element_type=jnp.float32)
    ss = jnp.sum(s8 * s8, axis=1, keepdims=True)
    br = bn.shape[0]
    z1 = jnp.zeros((br, 1), jnp.float32)
    srows_ref[...] = jnp.concatenate(
        [s8[:, 0:3], z1, ss, jnp.zeros((br, 3), jnp.float32)], axis=1)


def _projections(h2, a1, c1, Wh, bh, Ws8):
    br = 512
    grid = NP // br
    return pl.pallas_call(
        _proj_body,
        grid=grid,
        in_specs=[
            pl.BlockSpec((br, D), lambda i: (i, 0)),
            pl.BlockSpec((1, D), lambda i: (0, 0)),
            pl.BlockSpec((1, D), lambda i: (0, 0)),
            pl.BlockSpec((D, D), lambda i: (0, 0)),
            pl.BlockSpec((1, D), lambda i: (0, 0)),
            pl.BlockSpec((D, 8), lambda i: (0, 0)),
        ],
        out_specs=[
            pl.BlockSpec((br, D), lambda i: (i, 0)),
            pl.BlockSpec((br, D), lambda i: (i, 0)),
            pl.BlockSpec((br, 8), lambda i: (i, 0)),
        ],
        out_shape=[
            jax.ShapeDtypeStruct((NP, D), jnp.float32),
            jax.ShapeDtypeStruct((NP, D), jnp.float32),
            jax.ShapeDtypeStruct((NP, 8), jnp.float32),
        ],
        interpret=_IP,
    )(h2, a1, c1, Wh, bh, Ws8)


def _knn_body(srows_ref, sT_ref, idx_ref, d2_ref):
    i = pl.program_id(0)
    br = srows_ref.shape[0]
    d2 = (srows_ref[:, 4:5] + sT_ref[4:5, :]
          - 2.0 * jnp.dot(srows_ref[:, 0:4], sT_ref[0:4, :],
                          preferred_element_type=jnp.float32))
    rows = i * br + lax.broadcasted_iota(jnp.int32, (br, NP), 0)
    cols = lax.broadcasted_iota(jnp.int32, (br, NP), 1)
    d2 = jnp.where(cols == rows, BIG, d2)
    for t in range(K):
        j = jnp.argmin(d2, axis=1).astype(jnp.int32).reshape(br, 1)
        idx_ref[:, t:t + 1] = j
        if t < K - 1:
            d2_ref[...] = jnp.where(cols == j, BIG, d2)
            d2 = d2_ref[...]


def _knn(srows, sT):
    br = 256
    grid = NP // br
    return pl.pallas_call(
        _knn_body,
        grid=grid,
        in_specs=[
            pl.BlockSpec((br, 8), lambda i: (i, 0)),
            pl.BlockSpec((8, NP), lambda i: (0, 0)),
        ],
        out_specs=pl.BlockSpec((br, K), lambda i: (i, 0)),
        out_shape=jax.ShapeDtypeStruct((NP, K), jnp.int32),
        scratch_shapes=[pltpu.VMEM((br, NP), jnp.float32)],
        compiler_params=pltpu.CompilerParams(
            dimension_semantics=("parallel",)),
        interpret=_IP,
    )(srows, sT)


_NW = 32
_BPW = NP // _NW
_CH = 32
_NCH = _BPW // _CH


def _sc_agg_body(idx_hbm, stab_hbm, s16_hbm, hl_hbm, out_hbm,
                 xs_v, ys_v, zs_v, ss_v, idx_v, rows_v, s_v, agg_v, sem):
    wid = lax.axis_index("s") * 2 + lax.axis_index("c")
    pltpu.sync_copy(stab_hbm.at[0], xs_v)
    pltpu.sync_copy(stab_hbm.at[1], ys_v)
    pltpu.sync_copy(stab_hbm.at[2], zs_v)
    pltpu.sync_copy(stab_hbm.at[3], ss_v)

    def chunk_body(c, _):
        base = wid * _BPW + c * _CH
        pltpu.sync_copy(idx_hbm.at[pl.ds(base * K, _CH * K)], idx_v)
        pltpu.sync_copy(s16_hbm.at[pl.ds(base, _CH)], s_v)
        descs = [
            pltpu.async_copy(hl_hbm.at[idx_v.at[pl.ds(g * 128, 128)]],
                             rows_v.at[pl.ds(g * 128, 128)], sem)
            for g in range(_CH * K // 128)
        ]
        for dsc in descs:
            dsc.wait()

        def dst_body(di, _):
            svec = s_v[di, 0:16]
            si0, si1, si2, ssi = svec[0], svec[1], svec[2], svec[4]
            pots = []
            for off in (0, 16, 24):
                jg = idx_v[pl.ds(di * K + off, 16)]
                sjx = plsc.load_gather(xs_v, [jg])
                sjy = plsc.load_gather(ys_v, [jg])
                sjz = plsc.load_gather(zs_v, [jg])
                sjs = plsc.load_gather(ss_v, [jg])
                dd = ssi + sjs - 2.0 * (si0 * sjx + si1 * sjy + si2 * sjz)
                pots.append(jnp.exp(-jnp.maximum(dd, 0.0)))
            pa, pb, pc = pots
            z = jnp.zeros((16,), jnp.float32)
            neg = jnp.full((16,), -3.0e38, jnp.float32)
            m0, m1, x0, x1 = z, z, neg, neg
            for k in range(K):
                p = pa[k] if k < 16 else (pb[k - 16] if k < 32 else pc[k - 24])
                r = di * K + k
                f0 = p * rows_v[r, 0:16]
                f1 = p * rows_v[r, 16:32]
                m0 = m0 + f0
                m1 = m1 + f1
                x0 = jnp.maximum(x0, f0)
                x1 = jnp.maximum(x1, f1)
            agg_v[di, 0:16] = m0 / 40.0
            agg_v[di, 16:32] = m1 / 40.0
            agg_v[di, 32:48] = x0
            agg_v[di, 48:64] = x1
            return 0

        lax.fori_loop(0, _CH, dst_body, 0)
        pltpu.sync_copy(agg_v, out_hbm.at[pl.ds(base, _CH)])
        return 0

    lax.fori_loop(0, _NCH, chunk_body, 0)


def _sc_aggregate(idx_flat, stab, s16, hl):
    mesh = plsc.VectorSubcoreMesh(core_axis_name="c", subcore_axis_name="s")
    f = pl.kernel(
        _sc_agg_body,
        out_type=jax.ShapeDtypeStruct((NP, 2 * D), jnp.float32),
        mesh=mesh,
        scratch_types=[
            pltpu.VMEM((NP,), jnp.float32),
            pltpu.VMEM((NP,), jnp.float32),
            pltpu.VMEM((NP,), jnp.float32),
            pltpu.VMEM((NP,), jnp.float32),
            pltpu.VMEM((_CH * K,), jnp.int32),
            pltpu.VMEM((_CH * K, D), jnp.float32),
            pltpu.VMEM((_CH, 16), jnp.float32),
            pltpu.VMEM((_CH, 2 * D), jnp.float32),
            pltpu.SemaphoreType.DMA,
        ],
        compiler_params=pltpu.CompilerParams(use_tc_tiling_on_sc=False,
                                             needs_layout_passes=False),
    )
    return f(idx_flat, stab, s16, hl)


def _post_body(agg_ref, bn_ref, srows_ref, wla_ref, wlh_ref, bl_ref,
               wp1x_ref, wp1s_ref, wp1h_ref, bp1_ref, wp2_ref, bp2_ref,
               z2_ref, sums_ref):
    i = pl.program_id(0)
    xgn = (jnp.dot(agg_ref[...], wla_ref[...],
                   preferred_element_type=jnp.float32)
           + jnp.dot(bn_ref[...], wlh_ref[...],
                     preferred_element_type=jnp.float32)
           + bl_ref[...])
    z = (jnp.dot(xgn, wp1x_ref[...], preferred_element_type=jnp.float32)
         + jnp.dot(srows_ref[...], wp1s_ref[...],
                   preferred_element_type=jnp.float32)
         + jnp.dot(bn_ref[...], wp1h_ref[...],
                   preferred_element_type=jnp.float32)
         + bp1_ref[...])
    z = _elu(z)
    z = jnp.dot(z, wp2_ref[...], preferred_element_type=jnp.float32)
    z = _elu(z + bp2_ref[...])
    z2_ref[...] = z
    br = z.shape[0]
    rows = i * br + lax.broadcasted_iota(jnp.int32, (br, 1), 0)
    zm = jnp.where(rows < N, z, 0.0)

    @pl.when(i == 0)
    def _():
        sums_ref[...] = jnp.zeros_like(sums_ref)

    sums_ref[0:1, :] += jnp.sum(zm, axis=0, keepdims=True)
    sums_ref[1:2, :] += jnp.sum(zm * zm, axis=0, keepdims=True)


def _post_mlp(agg, bn, srows, Wla, Wlh, bl, Wp1x, Wp1s, Wp1h, bp1, Wp2, bp2):
    br = 512
    grid = NP // br
    return pl.pallas_call(
        _post_body,
        grid=grid,
        in_specs=[
            pl.BlockSpec((br, 2 * D), lambda i: (i, 0)),
            pl.BlockSpec((br, D), lambda i: (i, 0)),
            pl.BlockSpec((br, 8), lambda i: (i, 0)),
            pl.BlockSpec((2 * D, D), lambda i: (0, 0)),
            pl.BlockSpec((D, D), lambda i: (0, 0)),
            pl.BlockSpec((1, D), lambda i: (0, 0)),
            pl.BlockSpec((D, D), lambda i: (0, 0)),
            pl.BlockSpec((8, D), lambda i: (0, 0)),
            pl.BlockSpec((D, D), lambda i: (0, 0)),
            pl.BlockSpec((1, D), lambda i: (0, 0)),
            pl.BlockSpec((D, D), lambda i: (0, 0)),
            pl.BlockSpec((1, D), lambda i: (0, 0)),
        ],
        out_specs=[
            pl.BlockSpec((br, D), lambda i: (i, 0)),
            pl.BlockSpec((2, D), lambda i: (0, 0)),
        ],
        out_shape=[
            jax.ShapeDtypeStruct((NP, D), jnp.float32),
            jax.ShapeDtypeStruct((2, D), jnp.float32),
        ],
        interpret=_IP,
    )(agg, bn, srows, Wla, Wlh, bl, Wp1x, Wp1s, Wp1h, bp1, Wp2, bp2)


def _aff_body(z_ref, a_ref, c_ref, o_ref):
    o_ref[...] = z_ref[...] * a_ref[...] + c_ref[...]


def _affine(z2, a2, c2):
    br = 1024
    grid = NP // br
    return pl.pallas_call(
        _aff_body,
        grid=grid,
        in_specs=[
            pl.BlockSpec((br, D), lambda i: (i, 0)),
            pl.BlockSpec((1, D), lambda i: (0, 0)),
            pl.BlockSpec((1, D), lambda i: (0, 0)),
        ],
        out_specs=pl.BlockSpec((br, D), lambda i: (i, 0)),
        out_shape=jax.ShapeDtypeStruct((NP, D), jnp.float32),
        interpret=_IP,
    )(z2, a2, c2)


def kernel(x, batch, original_coords, step_count, num_layer,
           W_pre1, b_pre1, W_pre2, b_pre2, g1, be1,
           W_s, W_h, b_h, W_lin, b_lin,
           W_post1, b_post1, W_post2, b_post2, g2, be2):
    eps = 1e-5
    xp = jnp.pad(x, ((0, NP - N), (0, 0)))
    r1 = lambda v: v.reshape(1, D)

    h2, sums1 = _pre_mlp(xp, W_pre1, r1(b_pre1), W_pre2, r1(b_pre2))
    mu1 = sums1[0] / N
    var1 = sums1[1] / N - mu1 * mu1
    a1 = g1 / jnp.sqrt(var1 + eps)
    c1 = be1 - mu1 * a1

    Ws8 = jnp.pad(W_s, ((0, 0), (0, 5)))
    bn, hl, srows = _projections(h2, a1.reshape(1, D), c1.reshape(1, D),
                                 W_h, r1(b_h), Ws8)

    sT = srows[:, 0:8].T
    colv = jnp.arange(NP) < N
    sT = sT.at[4, :].set(jnp.where(colv, sT[4, :], 3.0e38))

    nn_idx = _knn(srows, sT)

    stab = jnp.stack([srows[:, 0], srows[:, 1], srows[:, 2], srows[:, 4]])
    s16 = jnp.pad(srows, ((0, 0), (0, 8)))
    agg = _sc_aggregate(nn_idx.reshape(-1), stab, s16, hl)

    Wla, Wlh = W_lin[:2 * D], W_lin[2 * D:]
    Wp1x = W_post1[0:D]
    Wp1s = jnp.pad(W_post1[D:D + 3], ((0, 5), (0, 0)))
    Wp1h = W_post1[D + 3:]
    z2, sums2 = _post_mlp(agg, bn, srows, Wla, Wlh, r1(b_lin),
                          Wp1x, Wp1s, Wp1h, r1(b_post1), W_post2,
                          r1(b_post2))
    mu2 = sums2[0] / N
    var2 = sums2[1] / N - mu2 * mu2
    a2 = g2 / jnp.sqrt(var2 + eps)
    c2 = be2 - mu2 * a2

    out = _affine(z2, a2.reshape(1, D), c2.reshape(1, D))
    return out[:N]
